# baseline (speedup 1.0000x reference)
import functools

import jax
import jax.numpy as jnp
from jax.experimental import pallas as pl
from jax.experimental.pallas import tpu as pltpu


def _dist_min_kernel(y_ref, xt_ref, b_ref, out_ref, tot_ref, acc_ref):
    j = pl.program_id(0)
    nk = pl.num_programs(0)

    @pl.when(j == 0)
    def _init():
        acc_ref[...] = jnp.full(acc_ref.shape, jnp.inf, acc_ref.dtype)

    y_blk = y_ref[...]
    a = jnp.concatenate([y_blk, y_blk * y_blk], axis=1)
    d = jnp.dot(a, b_ref[...],
                preferred_element_type=jnp.float32)
    m8 = jnp.min(d.reshape(-1, 8, d.shape[1]), axis=0)
    acc_ref[...] = jnp.minimum(acc_ref[...], m8)

    @pl.when(j == nk - 1)
    def _finish():
        xt = xt_ref[...]
        x2 = jnp.sum(xt * xt, axis=0, keepdims=True)
        r = jnp.min(acc_ref[...], axis=0, keepdims=True) + x2
        out_ref[...] = r
        tot_ref[...] = jnp.sum(r).reshape(1, 1)


@functools.partial(jax.jit, static_argnames=())
def kernel(x, y):
    q, dim = x.shape
    k = y.shape[0]
    kb = 2000
    nk = k // kb
    assert nk * kb == k

    xt = x.T
    y_bf = y.astype(jnp.bfloat16)
    b = jnp.concatenate(
        [(-2.0 * xt).astype(jnp.bfloat16),
         jnp.ones((dim, q), jnp.bfloat16)], axis=0)

    closest_row, tot = pl.pallas_call(
        _dist_min_kernel,
        grid=(nk,),
        in_specs=[
            pl.BlockSpec((kb, dim), lambda j: (j, 0)),
            pl.BlockSpec((dim, q), lambda j: (0, 0)),
            pl.BlockSpec((2 * dim, q), lambda j: (0, 0)),
        ],
        out_specs=[
            pl.BlockSpec((1, q), lambda j: (0, 0)),
            pl.BlockSpec((1, 1), lambda j: (0, 0)),
        ],
        out_shape=[
            jax.ShapeDtypeStruct((1, q), jnp.float32),
            jax.ShapeDtypeStruct((1, 1), jnp.float32),
        ],
        scratch_shapes=[pltpu.VMEM((8, q), jnp.float32)],
        compiler_params=pltpu.CompilerParams(
            dimension_semantics=("arbitrary",),
        ),
    )(y_bf, xt, b)

    return (tot.reshape(()), closest_row.reshape(q))


# speedup vs baseline: 1.0788x; 1.0788x over previous
import functools

import jax
import jax.numpy as jnp
from jax.experimental import pallas as pl
from jax.experimental.pallas import tpu as pltpu


def _dist_min_kernel(y_ref, xt_ref, b_ref, out_ref, tot_ref, acc_ref):
    j = pl.program_id(0)
    nk = pl.num_programs(0)

    @pl.when(j == 0)
    def _init():
        acc_ref[...] = jnp.full(acc_ref.shape, jnp.inf, acc_ref.dtype)

    y_blk = y_ref[...].astype(jnp.bfloat16)
    a = jnp.concatenate([y_blk, y_blk * y_blk], axis=1)
    d = jnp.dot(a, b_ref[...],
                preferred_element_type=jnp.float32)
    m8 = jnp.min(d.reshape(-1, 8, d.shape[1]), axis=0)
    acc_ref[...] = jnp.minimum(acc_ref[...], m8)

    @pl.when(j == nk - 1)
    def _finish():
        xt = xt_ref[...]
        x2 = jnp.sum(xt * xt, axis=0, keepdims=True)
        r = jnp.min(acc_ref[...], axis=0, keepdims=True) + x2
        out_ref[...] = r
        tot_ref[...] = jnp.sum(r).reshape(1, 1)


@functools.partial(jax.jit, static_argnames=())
def kernel(x, y):
    q, dim = x.shape
    k = y.shape[0]
    kb = 4000
    nk = k // kb
    assert nk * kb == k

    xt = x.T
    b = jnp.concatenate(
        [(-2.0 * xt).astype(jnp.bfloat16),
         jnp.ones((dim, q), jnp.bfloat16)], axis=0)

    closest_row, tot = pl.pallas_call(
        _dist_min_kernel,
        grid=(nk,),
        in_specs=[
            pl.BlockSpec((kb, dim), lambda j: (j, 0)),
            pl.BlockSpec((dim, q), lambda j: (0, 0)),
            pl.BlockSpec((2 * dim, q), lambda j: (0, 0)),
        ],
        out_specs=[
            pl.BlockSpec((1, q), lambda j: (0, 0)),
            pl.BlockSpec((1, 1), lambda j: (0, 0)),
        ],
        out_shape=[
            jax.ShapeDtypeStruct((1, q), jnp.float32),
            jax.ShapeDtypeStruct((1, 1), jnp.float32),
        ],
        scratch_shapes=[pltpu.VMEM((8, q), jnp.float32)],
        compiler_params=pltpu.CompilerParams(
            dimension_semantics=("arbitrary",),
        ),
    )(y, xt, b)

    return (tot.reshape(()), closest_row.reshape(q))


# speedup vs baseline: 1.0917x; 1.0120x over previous
import functools

import jax
import jax.numpy as jnp
from jax.experimental import pallas as pl
from jax.experimental.pallas import tpu as pltpu


def _dist_min_kernel(y_ref, xt_ref, b_ref, out_ref, tot_ref, acc_ref):
    j = pl.program_id(0)
    nk = pl.num_programs(0)

    @pl.when(j == 0)
    def _init():
        acc_ref[...] = jnp.full(acc_ref.shape, jnp.inf, acc_ref.dtype)

    y_blk = y_ref[...].astype(jnp.bfloat16)
    a = jnp.concatenate([y_blk, y_blk * y_blk], axis=1)
    d = jnp.dot(a, b_ref[...],
                preferred_element_type=jnp.float32)
    m8 = jnp.min(d.reshape(-1, 8, d.shape[1]), axis=0)
    acc_ref[...] = jnp.minimum(acc_ref[...], m8)

    @pl.when(j == nk - 1)
    def _finish():
        xt = xt_ref[...]
        x2 = jnp.sum(xt * xt, axis=0, keepdims=True)
        r = jnp.min(acc_ref[...], axis=0, keepdims=True) + x2
        out_ref[...] = r
        tot_ref[...] = jnp.sum(r).reshape(1, 1)


@functools.partial(jax.jit, static_argnames=())
def kernel(x, y):
    q, dim = x.shape
    k = y.shape[0]
    kb = 5000
    nk = k // kb
    assert nk * kb == k

    xt = x.T
    b = jnp.concatenate(
        [(-2.0 * xt).astype(jnp.bfloat16),
         jnp.ones((dim, q), jnp.bfloat16)], axis=0)

    closest_row, tot = pl.pallas_call(
        _dist_min_kernel,
        grid=(nk,),
        in_specs=[
            pl.BlockSpec((kb, dim), lambda j: (j, 0)),
            pl.BlockSpec((dim, q), lambda j: (0, 0)),
            pl.BlockSpec((2 * dim, q), lambda j: (0, 0)),
        ],
        out_specs=[
            pl.BlockSpec((1, q), lambda j: (0, 0)),
            pl.BlockSpec((1, 1), lambda j: (0, 0)),
        ],
        out_shape=[
            jax.ShapeDtypeStruct((1, q), jnp.float32),
            jax.ShapeDtypeStruct((1, 1), jnp.float32),
        ],
        scratch_shapes=[pltpu.VMEM((8, q), jnp.float32)],
        compiler_params=pltpu.CompilerParams(
            dimension_semantics=("arbitrary",),
        ),
    )(y, xt, b)

    return (tot.reshape(()), closest_row.reshape(q))
